# baseline (speedup 1.0000x reference)
import functools
import math

import jax
import jax.numpy as jnp
from jax.experimental import pallas as pl
from jax.experimental.pallas import tpu as pltpu

D_IN = 1024
D = 256
TD = 32
CAP = 100000
KWTA_K = 12
TAU = 0.2
TOPK = 32

BLK = 2048
NB = -(-CAP // BLK)
NEG = float("-inf")


def _rdot(a, b):
    return jax.lax.dot_general(
        a, b, (((1,), (1,)), ((), ())), preferred_element_type=jnp.float32
    )


def _retrieve_kernel(t_ref, feat_ref, w1_ref, b1_ref, w2_ref, b2_ref, tag_ref,
                     wm1a_ref, wm1b_ref, bm1_ref, wm2_ref, bm2_ref, k_ref,
                     vals_ref, idxs_ref, q_scr, log_scr):
    i = pl.program_id(0)

    @pl.when(i == 0)
    def _compute_query():
        f = feat_ref[...]
        h = jnp.maximum(_rdot(f, w1_ref[...]) + b1_ref[...], 0.0)
        x = _rdot(h, w2_ref[...]) + b2_ref[...] + tag_ref[...]

        lane = jax.lax.broadcasted_iota(jnp.int32, (1, TD), 1)
        pair = (lane // 2).astype(jnp.float32)
        div = jnp.exp(pair * (2.0 * (-math.log(10000.0) / TD)))
        ang = t_ref[0] * div
        pe = jnp.where(lane % 2 == 0, jnp.sin(ang), jnp.cos(ang))

        z1 = jnp.maximum(
            _rdot(x, wm1a_ref[...]) + _rdot(pe, wm1b_ref[...]) + bm1_ref[...],
            0.0)
        z = _rdot(z1, wm2_ref[...]) + bm2_ref[...]

        lane_d = jax.lax.broadcasted_iota(jnp.int32, (1, D), 1)

        def kbody(_, carry):
            zw, _ = carry
            m = jnp.max(zw)
            idx = jnp.min(jnp.where(zw == m, lane_d, D))
            zw = jnp.where(lane_d == idx, NEG, zw)
            return zw, m

        _, thresh = jax.lax.fori_loop(0, KWTA_K, kbody,
                                      (z, jnp.float32(0.0)))
        zm = jnp.where(z >= thresh, z, 0.0)
        nrm = jnp.sqrt(jnp.sum(zm * zm))
        q_scr[...] = zm / jnp.maximum(nrm, 1e-12)

    logits = _rdot(q_scr[...], k_ref[...]) / TAU
    log_scr[pl.ds(i, 1), :] = logits

    @pl.when(i == NB - 1)
    def _topk():
        row = jax.lax.broadcasted_iota(jnp.int32, (NB, BLK), 0)
        col = jax.lax.broadcasted_iota(jnp.int32, (NB, BLK), 1)
        gidx = row * BLK + col
        log_scr[...] = jnp.where(gidx < CAP, log_scr[...], NEG)
        sel_iota = jax.lax.broadcasted_iota(jnp.int32, (TOPK,), 0)

        def tbody(j, carry):
            vals, idxs = carry
            ll = log_scr[...]
            m = jnp.max(ll)
            gi = jnp.min(jnp.where(ll == m, gidx, NB * BLK))
            log_scr[...] = jnp.where(gidx == gi, NEG, ll)
            sel = sel_iota == j
            vals = jnp.where(sel, m, vals)
            idxs = jnp.where(sel, gi, idxs)
            return vals, idxs

        vals, idxs = jax.lax.fori_loop(
            0, TOPK, tbody,
            (jnp.full((TOPK,), NEG), jnp.zeros((TOPK,), jnp.int32)))
        vals_ref[...] = vals
        idxs_ref[...] = idxs


def _gather_kernel(idx_ref, vals_ref, v_ref, out_ref):
    i = pl.program_id(0)

    @pl.when(i == 0)
    def _init():
        out_ref[...] = jnp.zeros_like(out_ref)

    vals = vals_ref[...]
    m = jnp.max(vals)
    s = jnp.sum(jnp.exp(vals - m))
    sel = jax.lax.broadcasted_iota(jnp.int32, (TOPK,), 0) == i
    vi = jnp.max(jnp.where(sel, vals, NEG))
    wi = jnp.exp(vi - m) / s
    out_ref[...] += wi * v_ref[0]


@jax.jit
def kernel(features, W1, b1, W2, b2, tag, Wm1, bm1, Wm2, bm2, K, V, t):
    f2 = features.reshape(1, D_IN)
    b1r = b1.reshape(1, 2 * D)
    b2r = b2.reshape(1, D)
    tagr = tag.reshape(1, D)
    wm1a = Wm1[:, :D]
    wm1b = Wm1[:, D:]
    bm1r = bm1.reshape(1, D)
    bm2r = bm2.reshape(1, D)
    tr = t.reshape(1)

    vals, idxs = pl.pallas_call(
        _retrieve_kernel,
        grid=(NB,),
        in_specs=[
            pl.BlockSpec(memory_space=pltpu.SMEM),
            pl.BlockSpec((1, D_IN), lambda i: (0, 0)),
            pl.BlockSpec((2 * D, D_IN), lambda i: (0, 0)),
            pl.BlockSpec((1, 2 * D), lambda i: (0, 0)),
            pl.BlockSpec((D, 2 * D), lambda i: (0, 0)),
            pl.BlockSpec((1, D), lambda i: (0, 0)),
            pl.BlockSpec((1, D), lambda i: (0, 0)),
            pl.BlockSpec((D, D), lambda i: (0, 0)),
            pl.BlockSpec((D, TD), lambda i: (0, 0)),
            pl.BlockSpec((1, D), lambda i: (0, 0)),
            pl.BlockSpec((D, D), lambda i: (0, 0)),
            pl.BlockSpec((1, D), lambda i: (0, 0)),
            pl.BlockSpec((BLK, D), lambda i: (i, 0)),
        ],
        out_specs=[
            pl.BlockSpec((TOPK,), lambda i: (0,)),
            pl.BlockSpec((TOPK,), lambda i: (0,)),
        ],
        out_shape=[
            jax.ShapeDtypeStruct((TOPK,), jnp.float32),
            jax.ShapeDtypeStruct((TOPK,), jnp.int32),
        ],
        scratch_shapes=[
            pltpu.VMEM((1, D), jnp.float32),
            pltpu.VMEM((NB, BLK), jnp.float32),
        ],
    )(tr, f2, W1, b1r, W2, b2r, tagr, wm1a, wm1b, bm1r, Wm2, bm2r, K)

    out = pl.pallas_call(
        _gather_kernel,
        grid_spec=pltpu.PrefetchScalarGridSpec(
            num_scalar_prefetch=1,
            grid=(TOPK,),
            in_specs=[
                pl.BlockSpec((TOPK,), lambda i, idx: (0,)),
                pl.BlockSpec((1, 1, D), lambda i, idx: (idx[i], 0, 0)),
            ],
            out_specs=pl.BlockSpec((1, D), lambda i, idx: (0, 0)),
        ),
        out_shape=jax.ShapeDtypeStruct((1, D), jnp.float32),
    )(idxs, vals, V.reshape(CAP, 1, D))

    return out.reshape(D)


# speedup vs baseline: 4.4310x; 4.4310x over previous
import functools
import math

import jax
import jax.numpy as jnp
from jax.experimental import pallas as pl
from jax.experimental.pallas import tpu as pltpu

D_IN = 1024
D = 256
TD = 32
CAP = 100000
KWTA_K = 12
TAU = 0.2
TOPK = 32

BLK = 2048
NB = -(-CAP // BLK)
NEG = float("-inf")


def _rdot(a, b):
    return jax.lax.dot_general(
        a, b, (((1,), (1,)), ((), ())), preferred_element_type=jnp.float32
    )


def _retrieve_kernel(t_ref, feat_ref, w1_ref, b1_ref, w2_ref, b2_ref, tag_ref,
                     wm1a_ref, wm1b_ref, bm1_ref, wm2_ref, bm2_ref, k_ref,
                     v_hbm, out_ref, q_scr, log_scr, rows_scr, sem):
    i = pl.program_id(0)

    @pl.when(i == 0)
    def _compute_query():
        f = feat_ref[...]
        h = jnp.maximum(_rdot(f, w1_ref[...]) + b1_ref[...], 0.0)
        x = _rdot(h, w2_ref[...]) + b2_ref[...] + tag_ref[...]

        lane = jax.lax.broadcasted_iota(jnp.int32, (1, TD), 1)
        pair = (lane // 2).astype(jnp.float32)
        div = jnp.exp(pair * (2.0 * (-math.log(10000.0) / TD)))
        ang = t_ref[0] * div
        pe = jnp.where(lane % 2 == 0, jnp.sin(ang), jnp.cos(ang))

        z1 = jnp.maximum(
            _rdot(x, wm1a_ref[...]) + _rdot(pe, wm1b_ref[...]) + bm1_ref[...],
            0.0)
        z = _rdot(z1, wm2_ref[...]) + bm2_ref[...]

        lane_d = jax.lax.broadcasted_iota(jnp.int32, (1, D), 1)

        def kbody(_, carry):
            zw, _ = carry
            m = jnp.max(zw)
            idx = jnp.min(jnp.where(zw == m, lane_d, D))
            zw = jnp.where(lane_d == idx, NEG, zw)
            return zw, m

        _, thresh = jax.lax.fori_loop(0, KWTA_K, kbody,
                                      (z, jnp.float32(0.0)))
        zm = jnp.where(z >= thresh, z, 0.0)
        nrm = jnp.sqrt(jnp.sum(zm * zm))
        q_scr[...] = zm / jnp.maximum(nrm, 1e-12)

    logits = _rdot(q_scr[...], k_ref[...]) / TAU
    log_scr[pl.ds(i, 1), :] = logits

    @pl.when(i == NB - 1)
    def _topk_gather():
        row = jax.lax.broadcasted_iota(jnp.int32, (NB, BLK), 0)
        col = jax.lax.broadcasted_iota(jnp.int32, (NB, BLK), 1)
        gidx = row * BLK + col
        log_scr[...] = jnp.where(gidx < CAP, log_scr[...], NEG)
        sel_iota = jax.lax.broadcasted_iota(jnp.int32, (TOPK,), 0)

        def tbody(j, vals):
            ll = log_scr[...]
            m = jnp.max(ll)
            gi = jnp.min(jnp.where(ll == m, gidx, NB * BLK))
            log_scr[...] = jnp.where(gidx == gi, NEG, ll)
            pltpu.make_async_copy(
                v_hbm.at[pl.ds(gi, 1), :], rows_scr.at[pl.ds(j, 1), :], sem
            ).start()
            return jnp.where(sel_iota == j, m, vals)

        vals = jax.lax.fori_loop(0, TOPK, tbody, jnp.full((TOPK,), NEG))

        def wbody(j, c):
            pltpu.make_async_copy(
                v_hbm.at[pl.ds(0, 1), :], rows_scr.at[pl.ds(0, 1), :], sem
            ).wait()
            return c

        jax.lax.fori_loop(0, TOPK, wbody, 0)

        m = jnp.max(vals)
        e = jnp.exp(vals - m)
        w = (e / jnp.sum(e)).reshape(1, TOPK)
        out_ref[...] = jax.lax.dot_general(
            w, rows_scr[...], (((1,), (0,)), ((), ())),
            preferred_element_type=jnp.float32)


@jax.jit
def kernel(features, W1, b1, W2, b2, tag, Wm1, bm1, Wm2, bm2, K, V, t):
    f2 = features.reshape(1, D_IN)
    b1r = b1.reshape(1, 2 * D)
    b2r = b2.reshape(1, D)
    tagr = tag.reshape(1, D)
    wm1a = Wm1[:, :D]
    wm1b = Wm1[:, D:]
    bm1r = bm1.reshape(1, D)
    bm2r = bm2.reshape(1, D)
    tr = t.reshape(1)

    out = pl.pallas_call(
        _retrieve_kernel,
        grid=(NB,),
        in_specs=[
            pl.BlockSpec(memory_space=pltpu.SMEM),
            pl.BlockSpec((1, D_IN), lambda i: (0, 0)),
            pl.BlockSpec((2 * D, D_IN), lambda i: (0, 0)),
            pl.BlockSpec((1, 2 * D), lambda i: (0, 0)),
            pl.BlockSpec((D, 2 * D), lambda i: (0, 0)),
            pl.BlockSpec((1, D), lambda i: (0, 0)),
            pl.BlockSpec((1, D), lambda i: (0, 0)),
            pl.BlockSpec((D, D), lambda i: (0, 0)),
            pl.BlockSpec((D, TD), lambda i: (0, 0)),
            pl.BlockSpec((1, D), lambda i: (0, 0)),
            pl.BlockSpec((D, D), lambda i: (0, 0)),
            pl.BlockSpec((1, D), lambda i: (0, 0)),
            pl.BlockSpec((BLK, D), lambda i: (i, 0)),
            pl.BlockSpec(memory_space=pl.ANY),
        ],
        out_specs=pl.BlockSpec((1, D), lambda i: (0, 0)),
        out_shape=jax.ShapeDtypeStruct((1, D), jnp.float32),
        scratch_shapes=[
            pltpu.VMEM((1, D), jnp.float32),
            pltpu.VMEM((NB, BLK), jnp.float32),
            pltpu.VMEM((TOPK, D), jnp.float32),
            pltpu.SemaphoreType.DMA,
        ],
    )(tr, f2, W1, b1r, W2, b2r, tagr, wm1a, wm1b, bm1r, Wm2, bm2r, K, V)

    return out.reshape(D)


# speedup vs baseline: 5.2506x; 1.1850x over previous
import functools
import math

import jax
import jax.numpy as jnp
from jax.experimental import pallas as pl
from jax.experimental.pallas import tpu as pltpu

D_IN = 1024
D = 256
TD = 32
CAP = 100000
KWTA_K = 12
TAU = 0.2
TOPK = 32

BLK = 4096
NB = -(-CAP // BLK)
NEG = float("-inf")


def _rdot(a, b):
    return jax.lax.dot_general(
        a, b, (((1,), (1,)), ((), ())), preferred_element_type=jnp.float32
    )


def _retrieve_kernel(t_ref, feat_ref, w1_ref, b1_ref, w2_ref, b2_ref, tag_ref,
                     wm1a_ref, wm1b_ref, bm1_ref, wm2_ref, bm2_ref, k_ref,
                     v_hbm, out_ref, q_scr, log_scr, rows_scr, sem):
    i = pl.program_id(0)

    @pl.when(i == 0)
    def _compute_query():
        f = feat_ref[...]
        h = jnp.maximum(_rdot(f, w1_ref[...]) + b1_ref[...], 0.0)
        x = _rdot(h, w2_ref[...]) + b2_ref[...] + tag_ref[...]

        lane = jax.lax.broadcasted_iota(jnp.int32, (1, TD), 1)
        pair = (lane // 2).astype(jnp.float32)
        div = jnp.exp(pair * (2.0 * (-math.log(10000.0) / TD)))
        ang = t_ref[0] * div
        pe = jnp.where(lane % 2 == 0, jnp.sin(ang), jnp.cos(ang))

        z1 = jnp.maximum(
            _rdot(x, wm1a_ref[...]) + _rdot(pe, wm1b_ref[...]) + bm1_ref[...],
            0.0)
        z = _rdot(z1, wm2_ref[...]) + bm2_ref[...]

        lane_d = jax.lax.broadcasted_iota(jnp.int32, (1, D), 1)

        def kbody(_, carry):
            zw, _ = carry
            m = jnp.max(zw)
            idx = jnp.min(jnp.where(zw == m, lane_d, D))
            zw = jnp.where(lane_d == idx, NEG, zw)
            return zw, m

        _, thresh = jax.lax.fori_loop(0, KWTA_K, kbody,
                                      (z, jnp.float32(0.0)))
        zm = jnp.where(z >= thresh, z, 0.0)
        nrm = jnp.sqrt(jnp.sum(zm * zm))
        q_scr[...] = zm / jnp.maximum(nrm, 1e-12)

    logits = _rdot(q_scr[...], k_ref[...]) / TAU
    log_scr[pl.ds(i, 1), :] = logits

    @pl.when(i == NB - 1)
    def _topk_gather():
        row = jax.lax.broadcasted_iota(jnp.int32, (NB, BLK), 0)
        col = jax.lax.broadcasted_iota(jnp.int32, (NB, BLK), 1)
        gidx = row * BLK + col
        log_scr[...] = jnp.where(gidx < CAP, log_scr[...], NEG)
        sel_iota = jax.lax.broadcasted_iota(jnp.int32, (TOPK,), 0)

        def tbody(j, vals):
            ll = log_scr[...]
            m = jnp.max(ll)
            gi = jnp.min(jnp.where(ll == m, gidx, NB * BLK))
            log_scr[...] = jnp.where(gidx == gi, NEG, ll)
            pltpu.make_async_copy(
                v_hbm.at[pl.ds(gi, 1), :], rows_scr.at[pl.ds(j, 1), :], sem
            ).start()
            return jnp.where(sel_iota == j, m, vals)

        vals = jax.lax.fori_loop(0, TOPK, tbody, jnp.full((TOPK,), NEG))

        def wbody(j, c):
            pltpu.make_async_copy(
                v_hbm.at[pl.ds(0, 1), :], rows_scr.at[pl.ds(0, 1), :], sem
            ).wait()
            return c

        jax.lax.fori_loop(0, TOPK, wbody, 0)

        m = jnp.max(vals)
        e = jnp.exp(vals - m)
        w = (e / jnp.sum(e)).reshape(1, TOPK)
        out_ref[...] = jax.lax.dot_general(
            w, rows_scr[...], (((1,), (0,)), ((), ())),
            preferred_element_type=jnp.float32)


@jax.jit
def kernel(features, W1, b1, W2, b2, tag, Wm1, bm1, Wm2, bm2, K, V, t):
    f2 = features.reshape(1, D_IN)
    b1r = b1.reshape(1, 2 * D)
    b2r = b2.reshape(1, D)
    tagr = tag.reshape(1, D)
    wm1a = Wm1[:, :D]
    wm1b = Wm1[:, D:]
    bm1r = bm1.reshape(1, D)
    bm2r = bm2.reshape(1, D)
    tr = t.reshape(1)

    out = pl.pallas_call(
        _retrieve_kernel,
        grid=(NB,),
        in_specs=[
            pl.BlockSpec(memory_space=pltpu.SMEM),
            pl.BlockSpec((1, D_IN), lambda i: (0, 0)),
            pl.BlockSpec((2 * D, D_IN), lambda i: (0, 0)),
            pl.BlockSpec((1, 2 * D), lambda i: (0, 0)),
            pl.BlockSpec((D, 2 * D), lambda i: (0, 0)),
            pl.BlockSpec((1, D), lambda i: (0, 0)),
            pl.BlockSpec((1, D), lambda i: (0, 0)),
            pl.BlockSpec((D, D), lambda i: (0, 0)),
            pl.BlockSpec((D, TD), lambda i: (0, 0)),
            pl.BlockSpec((1, D), lambda i: (0, 0)),
            pl.BlockSpec((D, D), lambda i: (0, 0)),
            pl.BlockSpec((1, D), lambda i: (0, 0)),
            pl.BlockSpec((BLK, D), lambda i: (i, 0)),
            pl.BlockSpec(memory_space=pl.ANY),
        ],
        out_specs=pl.BlockSpec((1, D), lambda i: (0, 0)),
        out_shape=jax.ShapeDtypeStruct((1, D), jnp.float32),
        scratch_shapes=[
            pltpu.VMEM((1, D), jnp.float32),
            pltpu.VMEM((NB, BLK), jnp.float32),
            pltpu.VMEM((TOPK, D), jnp.float32),
            pltpu.SemaphoreType.DMA,
        ],
    )(tr, f2, W1, b1r, W2, b2r, tagr, wm1a, wm1b, bm1r, Wm2, bm2r, K, V)

    return out.reshape(D)


# speedup vs baseline: 5.6747x; 1.0808x over previous
import functools
import math

import jax
import jax.numpy as jnp
from jax.experimental import pallas as pl
from jax.experimental.pallas import tpu as pltpu

D_IN = 1024
D = 256
TD = 32
CAP = 100000
KWTA_K = 12
TAU = 0.2
TOPK = 32

BLK = 8192
NB = -(-CAP // BLK)
NEG = float("-inf")


def _rdot(a, b):
    return jax.lax.dot_general(
        a, b, (((1,), (1,)), ((), ())), preferred_element_type=jnp.float32
    )


def _retrieve_kernel(t_ref, feat_ref, w1_ref, b1_ref, w2_ref, b2_ref, tag_ref,
                     wm1a_ref, wm1b_ref, bm1_ref, wm2_ref, bm2_ref, k_ref,
                     v_hbm, out_ref, q_scr, log_scr, rows_scr, sem):
    i = pl.program_id(0)

    @pl.when(i == 0)
    def _compute_query():
        f = feat_ref[...]
        h = jnp.maximum(_rdot(f, w1_ref[...]) + b1_ref[...], 0.0)
        x = _rdot(h, w2_ref[...]) + b2_ref[...] + tag_ref[...]

        lane = jax.lax.broadcasted_iota(jnp.int32, (1, TD), 1)
        pair = (lane // 2).astype(jnp.float32)
        div = jnp.exp(pair * (2.0 * (-math.log(10000.0) / TD)))
        ang = t_ref[0] * div
        pe = jnp.where(lane % 2 == 0, jnp.sin(ang), jnp.cos(ang))

        z1 = jnp.maximum(
            _rdot(x, wm1a_ref[...]) + _rdot(pe, wm1b_ref[...]) + bm1_ref[...],
            0.0)
        z = _rdot(z1, wm2_ref[...]) + bm2_ref[...]

        lane_d = jax.lax.broadcasted_iota(jnp.int32, (1, D), 1)

        def kbody(_, carry):
            zw, _ = carry
            m = jnp.max(zw)
            idx = jnp.min(jnp.where(zw == m, lane_d, D))
            zw = jnp.where(lane_d == idx, NEG, zw)
            return zw, m

        _, thresh = jax.lax.fori_loop(0, KWTA_K, kbody,
                                      (z, jnp.float32(0.0)))
        zm = jnp.where(z >= thresh, z, 0.0)
        nrm = jnp.sqrt(jnp.sum(zm * zm))
        q_scr[...] = zm / jnp.maximum(nrm, 1e-12)

    logits = _rdot(q_scr[...], k_ref[...]) / TAU
    log_scr[pl.ds(i, 1), :] = logits

    @pl.when(i == NB - 1)
    def _topk_gather():
        row = jax.lax.broadcasted_iota(jnp.int32, (NB, BLK), 0)
        col = jax.lax.broadcasted_iota(jnp.int32, (NB, BLK), 1)
        gidx = row * BLK + col
        log_scr[...] = jnp.where(gidx < CAP, log_scr[...], NEG)
        sel_iota = jax.lax.broadcasted_iota(jnp.int32, (TOPK,), 0)

        def tbody(j, vals):
            ll = log_scr[...]
            m = jnp.max(ll)
            gi = jnp.min(jnp.where(ll == m, gidx, NB * BLK))
            log_scr[...] = jnp.where(gidx == gi, NEG, ll)
            pltpu.make_async_copy(
                v_hbm.at[pl.ds(gi, 1), :], rows_scr.at[pl.ds(j, 1), :], sem
            ).start()
            return jnp.where(sel_iota == j, m, vals)

        vals = jax.lax.fori_loop(0, TOPK, tbody, jnp.full((TOPK,), NEG))

        def wbody(j, c):
            pltpu.make_async_copy(
                v_hbm.at[pl.ds(0, 1), :], rows_scr.at[pl.ds(0, 1), :], sem
            ).wait()
            return c

        jax.lax.fori_loop(0, TOPK, wbody, 0)

        m = jnp.max(vals)
        e = jnp.exp(vals - m)
        w = (e / jnp.sum(e)).reshape(1, TOPK)
        out_ref[...] = jax.lax.dot_general(
            w, rows_scr[...], (((1,), (0,)), ((), ())),
            preferred_element_type=jnp.float32)


@jax.jit
def kernel(features, W1, b1, W2, b2, tag, Wm1, bm1, Wm2, bm2, K, V, t):
    f2 = features.reshape(1, D_IN)
    b1r = b1.reshape(1, 2 * D)
    b2r = b2.reshape(1, D)
    tagr = tag.reshape(1, D)
    wm1a = Wm1[:, :D]
    wm1b = Wm1[:, D:]
    bm1r = bm1.reshape(1, D)
    bm2r = bm2.reshape(1, D)
    tr = t.reshape(1)

    out = pl.pallas_call(
        _retrieve_kernel,
        grid=(NB,),
        in_specs=[
            pl.BlockSpec(memory_space=pltpu.SMEM),
            pl.BlockSpec((1, D_IN), lambda i: (0, 0)),
            pl.BlockSpec((2 * D, D_IN), lambda i: (0, 0)),
            pl.BlockSpec((1, 2 * D), lambda i: (0, 0)),
            pl.BlockSpec((D, 2 * D), lambda i: (0, 0)),
            pl.BlockSpec((1, D), lambda i: (0, 0)),
            pl.BlockSpec((1, D), lambda i: (0, 0)),
            pl.BlockSpec((D, D), lambda i: (0, 0)),
            pl.BlockSpec((D, TD), lambda i: (0, 0)),
            pl.BlockSpec((1, D), lambda i: (0, 0)),
            pl.BlockSpec((D, D), lambda i: (0, 0)),
            pl.BlockSpec((1, D), lambda i: (0, 0)),
            pl.BlockSpec((BLK, D), lambda i: (i, 0)),
            pl.BlockSpec(memory_space=pl.ANY),
        ],
        out_specs=pl.BlockSpec((1, D), lambda i: (0, 0)),
        out_shape=jax.ShapeDtypeStruct((1, D), jnp.float32),
        scratch_shapes=[
            pltpu.VMEM((1, D), jnp.float32),
            pltpu.VMEM((NB, BLK), jnp.float32),
            pltpu.VMEM((TOPK, D), jnp.float32),
            pltpu.SemaphoreType.DMA,
        ],
    )(tr, f2, W1, b1r, W2, b2r, tagr, wm1a, wm1b, bm1r, Wm2, bm2r, K, V)

    return out.reshape(D)


# speedup vs baseline: 5.8106x; 1.0240x over previous
import functools
import math

import jax
import jax.numpy as jnp
from jax.experimental import pallas as pl
from jax.experimental.pallas import tpu as pltpu

D_IN = 1024
D = 256
TD = 32
CAP = 100000
KWTA_K = 12
TAU = 0.2
TOPK = 32

BLK = 16384
NB = -(-CAP // BLK)
NEG = float("-inf")


def _rdot(a, b):
    return jax.lax.dot_general(
        a, b, (((1,), (1,)), ((), ())), preferred_element_type=jnp.float32
    )


def _retrieve_kernel(t_ref, feat_ref, w1_ref, b1_ref, w2_ref, b2_ref, tag_ref,
                     wm1a_ref, wm1b_ref, bm1_ref, wm2_ref, bm2_ref, k_ref,
                     v_hbm, out_ref, q_scr, log_scr, rows_scr, sem):
    i = pl.program_id(0)

    @pl.when(i == 0)
    def _compute_query():
        f = feat_ref[...]
        h = jnp.maximum(_rdot(f, w1_ref[...]) + b1_ref[...], 0.0)
        x = _rdot(h, w2_ref[...]) + b2_ref[...] + tag_ref[...]

        lane = jax.lax.broadcasted_iota(jnp.int32, (1, TD), 1)
        pair = (lane // 2).astype(jnp.float32)
        div = jnp.exp(pair * (2.0 * (-math.log(10000.0) / TD)))
        ang = t_ref[0] * div
        pe = jnp.where(lane % 2 == 0, jnp.sin(ang), jnp.cos(ang))

        z1 = jnp.maximum(
            _rdot(x, wm1a_ref[...]) + _rdot(pe, wm1b_ref[...]) + bm1_ref[...],
            0.0)
        z = _rdot(z1, wm2_ref[...]) + bm2_ref[...]

        lane_d = jax.lax.broadcasted_iota(jnp.int32, (1, D), 1)

        def kbody(_, carry):
            zw, _ = carry
            m = jnp.max(zw)
            idx = jnp.min(jnp.where(zw == m, lane_d, D))
            zw = jnp.where(lane_d == idx, NEG, zw)
            return zw, m

        _, thresh = jax.lax.fori_loop(0, KWTA_K, kbody,
                                      (z, jnp.float32(0.0)))
        zm = jnp.where(z >= thresh, z, 0.0)
        nrm = jnp.sqrt(jnp.sum(zm * zm))
        q_scr[...] = zm / jnp.maximum(nrm, 1e-12)

    logits = _rdot(q_scr[...], k_ref[...]) / TAU
    log_scr[pl.ds(i, 1), :] = logits

    @pl.when(i == NB - 1)
    def _topk_gather():
        row = jax.lax.broadcasted_iota(jnp.int32, (NB, BLK), 0)
        col = jax.lax.broadcasted_iota(jnp.int32, (NB, BLK), 1)
        gidx = row * BLK + col
        log_scr[...] = jnp.where(gidx < CAP, log_scr[...], NEG)
        sel_iota = jax.lax.broadcasted_iota(jnp.int32, (TOPK,), 0)

        def tbody(j, vals):
            ll = log_scr[...]
            m = jnp.max(ll)
            gi = jnp.min(jnp.where(ll == m, gidx, NB * BLK))
            log_scr[...] = jnp.where(gidx == gi, NEG, ll)
            pltpu.make_async_copy(
                v_hbm.at[pl.ds(gi, 1), :], rows_scr.at[pl.ds(j, 1), :], sem
            ).start()
            return jnp.where(sel_iota == j, m, vals)

        vals = jax.lax.fori_loop(0, TOPK, tbody, jnp.full((TOPK,), NEG))

        def wbody(j, c):
            pltpu.make_async_copy(
                v_hbm.at[pl.ds(0, 1), :], rows_scr.at[pl.ds(0, 1), :], sem
            ).wait()
            return c

        jax.lax.fori_loop(0, TOPK, wbody, 0)

        m = jnp.max(vals)
        e = jnp.exp(vals - m)
        w = (e / jnp.sum(e)).reshape(1, TOPK)
        out_ref[...] = jax.lax.dot_general(
            w, rows_scr[...], (((1,), (0,)), ((), ())),
            preferred_element_type=jnp.float32)


@jax.jit
def kernel(features, W1, b1, W2, b2, tag, Wm1, bm1, Wm2, bm2, K, V, t):
    f2 = features.reshape(1, D_IN)
    b1r = b1.reshape(1, 2 * D)
    b2r = b2.reshape(1, D)
    tagr = tag.reshape(1, D)
    wm1a = Wm1[:, :D]
    wm1b = Wm1[:, D:]
    bm1r = bm1.reshape(1, D)
    bm2r = bm2.reshape(1, D)
    tr = t.reshape(1)

    out = pl.pallas_call(
        _retrieve_kernel,
        grid=(NB,),
        in_specs=[
            pl.BlockSpec(memory_space=pltpu.SMEM),
            pl.BlockSpec((1, D_IN), lambda i: (0, 0)),
            pl.BlockSpec((2 * D, D_IN), lambda i: (0, 0)),
            pl.BlockSpec((1, 2 * D), lambda i: (0, 0)),
            pl.BlockSpec((D, 2 * D), lambda i: (0, 0)),
            pl.BlockSpec((1, D), lambda i: (0, 0)),
            pl.BlockSpec((1, D), lambda i: (0, 0)),
            pl.BlockSpec((D, D), lambda i: (0, 0)),
            pl.BlockSpec((D, TD), lambda i: (0, 0)),
            pl.BlockSpec((1, D), lambda i: (0, 0)),
            pl.BlockSpec((D, D), lambda i: (0, 0)),
            pl.BlockSpec((1, D), lambda i: (0, 0)),
            pl.BlockSpec((BLK, D), lambda i: (i, 0)),
            pl.BlockSpec(memory_space=pl.ANY),
        ],
        out_specs=pl.BlockSpec((1, D), lambda i: (0, 0)),
        out_shape=jax.ShapeDtypeStruct((1, D), jnp.float32),
        scratch_shapes=[
            pltpu.VMEM((1, D), jnp.float32),
            pltpu.VMEM((NB, BLK), jnp.float32),
            pltpu.VMEM((TOPK, D), jnp.float32),
            pltpu.SemaphoreType.DMA,
        ],
    )(tr, f2, W1, b1r, W2, b2r, tagr, wm1a, wm1b, bm1r, Wm2, bm2r, K, V)

    return out.reshape(D)
